# baseline (speedup 1.0000x reference)
import functools

import jax
import jax.numpy as jnp
from jax import lax
from jax.experimental import pallas as pl
from jax.experimental.pallas import tpu as pltpu
from jax.experimental.pallas import tpu_sc as plsc

NC, NS = 2, 16
NW = NC * NS
B, QL, CL, D = 1024, 20, 200, 64
DP = 128
TOK = QL + CL
TOKP = 224
B_PER_W = B // NW
NBUF = 4
OUTER = B_PER_W // NBUF


@functools.cache
def _build_gather_kernel():
    mesh = plsc.VectorSubcoreMesh(core_axis_name="c", subcore_axis_name="s")

    @functools.partial(
        pl.kernel,
        out_type=jax.ShapeDtypeStruct((B, TOKP, DP), jnp.float32),
        mesh=mesh,
        scratch_types=[
            pltpu.VMEM((B_PER_W, TOK), jnp.int32),
            pltpu.VMEM((NBUF, TOKP, DP), jnp.float32),
            pltpu.SemaphoreType.DMA((NBUF,)),
            pltpu.SemaphoreType.DMA((NBUF,)),
        ],
        compiler_params=pltpu.CompilerParams(use_tc_tiling_on_sc=False),
    )
    def _gather_kernel(idx_hbm, table_hbm, out_hbm, idx_v, bufs, gsem, wsem):
        wid = lax.axis_index("s") * NC + lax.axis_index("c")
        b0 = wid * B_PER_W
        pltpu.sync_copy(idx_hbm.at[pl.ds(b0, B_PER_W)], idx_v)

        def outer(o, carry):
            base = o * NBUF
            gathers = []
            for s in range(NBUF):
                @pl.when(o > 0)
                def _(s=s):
                    pltpu.make_async_copy(
                        bufs.at[s], out_hbm.at[0], wsem.at[s]
                    ).wait()

                bl = base + s
                gathers.append(
                    pltpu.async_copy(
                        table_hbm.at[idx_v.at[bl, pl.ds(0, 128)]],
                        bufs.at[s, pl.ds(0, 128)],
                        gsem.at[s],
                    )
                )
                gathers.append(
                    pltpu.async_copy(
                        table_hbm.at[idx_v.at[bl, pl.ds(128, TOK - 128)]],
                        bufs.at[s, pl.ds(128, TOK - 128)],
                        gsem.at[s],
                    )
                )
            for s in range(NBUF):
                gathers[2 * s].wait()
                gathers[2 * s + 1].wait()
                pltpu.async_copy(
                    bufs.at[s], out_hbm.at[b0 + base + s], wsem.at[s]
                )
            return carry

        lax.fori_loop(0, OUTER, outer, 0)
        for s in range(NBUF):
            pltpu.make_async_copy(
                bufs.at[s], out_hbm.at[0], wsem.at[s]
            ).wait()

    return _gather_kernel


def kernel(table, question_words, context_words):
    idx = jnp.concatenate(
        [question_words.astype(jnp.int32), context_words.astype(jnp.int32)],
        axis=1,
    )
    table_p = jnp.concatenate([table, table], axis=1)
    out_p = _build_gather_kernel()(idx, table_p)
    return out_p[:, :TOK, :D]


# speedup vs baseline: 1.2187x; 1.2187x over previous
import functools

import jax
import jax.numpy as jnp
from jax import lax
from jax.experimental import pallas as pl
from jax.experimental.pallas import tpu as pltpu
from jax.experimental.pallas import tpu_sc as plsc

NC, NS = 2, 16
NW = NC * NS
B, QL, CL, D = 1024, 20, 200, 64
DP = 128
TOK = QL + CL
TOKP = 224
B_PER_W = B // NW
NBUF = 4
OUTER = B_PER_W // NBUF


@functools.cache
def _build_gather_kernel():
    mesh = plsc.VectorSubcoreMesh(core_axis_name="c", subcore_axis_name="s")

    @functools.partial(
        pl.kernel,
        out_type=jax.ShapeDtypeStruct((B, TOKP, DP), jnp.float32),
        mesh=mesh,
        scratch_types=[
            pltpu.VMEM((B_PER_W, TOK), jnp.int32),
            pltpu.VMEM((NBUF, TOKP, DP), jnp.float32),
            pltpu.SemaphoreType.DMA((NBUF,)),
            pltpu.SemaphoreType.DMA((NBUF,)),
        ],
        compiler_params=pltpu.CompilerParams(use_tc_tiling_on_sc=False),
    )
    def _gather_kernel(idx_hbm, table_hbm, out_hbm, idx_v, bufs, gsem, wsem):
        wid = lax.axis_index("s") * NC + lax.axis_index("c")
        b0 = wid * B_PER_W
        pltpu.sync_copy(idx_hbm.at[pl.ds(b0, B_PER_W)], idx_v)

        def outer(o, carry):
            base = o * NBUF
            gathers = []
            for s in range(NBUF):
                @pl.when(o > 0)
                def _(s=s):
                    pltpu.make_async_copy(
                        bufs.at[s], out_hbm.at[0], wsem.at[s]
                    ).wait()

                bl = base + s
                gathers.append(
                    pltpu.async_copy(
                        table_hbm.at[idx_v.at[bl, pl.ds(0, 128)]],
                        bufs.at[s, pl.ds(0, 128)],
                        gsem.at[s],
                    )
                )
                gathers.append(
                    pltpu.async_copy(
                        table_hbm.at[idx_v.at[bl, pl.ds(128, TOK - 128)]],
                        bufs.at[s, pl.ds(128, TOK - 128)],
                        gsem.at[s],
                    )
                )
            for s in range(NBUF):
                gathers[2 * s].wait()
                gathers[2 * s + 1].wait()
                pltpu.async_copy(
                    bufs.at[s], out_hbm.at[b0 + base + s], wsem.at[s]
                )
            return carry

        lax.fori_loop(0, OUTER, outer, 0)
        for s in range(NBUF):
            pltpu.make_async_copy(
                bufs.at[s], out_hbm.at[0], wsem.at[s]
            ).wait()

    return _gather_kernel


def kernel(table, question_words, context_words):
    idx = jnp.concatenate(
        [question_words.astype(jnp.int32), context_words.astype(jnp.int32)],
        axis=1,
    )
    table_p = jnp.pad(table, ((0, 0), (0, DP - D)))
    out_p = _build_gather_kernel()(idx, table_p)
    return out_p[:, :TOK, :D]
